# baseline (speedup 1.0000x reference)
import functools

import jax
import jax.numpy as jnp
from jax import lax
from jax.experimental import pallas as pl
from jax.experimental.pallas import tpu as pltpu
from jax.experimental.pallas import tpu_sc as plsc


def _sc_agg_call(x2, src2, dst, zeros_c, ones_c,
                 n_pad, rows_per_tile, chunk, n_chunks, e):
  e_per_tile = chunk * n_chunks
  stripe = 64
  n_stripes = rows_per_tile // stripe
  chunk2 = 40
  n_chunks2 = (e // 2) // (16 * chunk2)

  def body(x2_ref, src2_ref, dst3_ref, dst_ref, zeros_ref, ones_ref,
           out_ref, deg_out_ref, agg_s, ib, db, dst2_v, rows_v,
           dst2_w, rows_w, sem, sem2):
    c = lax.axis_index("c")
    s = lax.axis_index("s")

    rbase = s * rows_per_tile
    for k in range(n_stripes):
      pltpu.sync_copy(zeros_ref, agg_s.at[pl.ds(rbase + k * stripe, stripe)])
    plsc.subcore_barrier()

    cbase = s * n_chunks
    n_chunk_rows = n_chunks * 16

    def pair_body(k, _):
      row_a = cbase + 2 * k
      pltpu.sync_copy(src2_ref.at[pl.ds(c * n_chunk_rows + row_a, 2)], ib)
      pltpu.sync_copy(dst3_ref.at[pl.ds(row_a, 2)], db)
      ga = pltpu.async_copy(x2_ref.at[ib.at[0, 0]], rows_v, sem)
      gb = pltpu.async_copy(x2_ref.at[ib.at[1, 0]], rows_w, sem2)
      ga.wait()
      pltpu.sync_copy(rows_v, agg_s.at[db.at[0, 0]], add=True)
      gb.wait()
      pltpu.sync_copy(rows_w, agg_s.at[db.at[1, 0]], add=True)
      return 0
    lax.fori_loop(0, n_chunks // 2, pair_body, 0)
    row_t = cbase + n_chunks - 1
    pltpu.sync_copy(src2_ref.at[pl.ds(c * n_chunk_rows + row_t, 1)],
                    ib.at[pl.ds(0, 1)])
    pltpu.sync_copy(dst3_ref.at[pl.ds(row_t, 1)], db.at[pl.ds(0, 1)])
    pltpu.async_copy(x2_ref.at[ib.at[0, 0]], rows_v, sem).wait()
    pltpu.sync_copy(rows_v, agg_s.at[db.at[0, 0]], add=True)
    plsc.subcore_barrier()

    for k in range(n_stripes):
      pltpu.sync_copy(agg_s.at[pl.ds(rbase + k * stripe, stripe)],
                      out_ref.at[c, pl.ds(rbase + k * stripe, stripe)])
    plsc.subcore_barrier()

    for k in range(n_stripes):
      pltpu.sync_copy(zeros_ref, agg_s.at[pl.ds(rbase + k * stripe, stripe)])
    pltpu.sync_copy(ones_ref, rows_v.at[pl.ds(0, chunk2)])
    plsc.subcore_barrier()

    ebase2 = (c * 16 + s) * (chunk2 * n_chunks2)

    def pair2_body(k, _):
      off_a = ebase2 + (2 * k) * chunk2
      off_b = off_a + chunk2
      pltpu.sync_copy(dst_ref.at[pl.ds(off_a, chunk2)], dst2_v)
      sa = pltpu.async_copy(rows_v.at[pl.ds(0, chunk2)],
                            agg_s.at[dst2_v], sem, add=True)
      pltpu.sync_copy(dst_ref.at[pl.ds(off_b, chunk2)], dst2_w)
      sa.wait()
      sb = pltpu.async_copy(rows_v.at[pl.ds(0, chunk2)],
                            agg_s.at[dst2_w], sem2, add=True)
      sb.wait()
      return 0
    lax.fori_loop(0, n_chunks2 // 2, pair2_body, 0)
    off_t2 = ebase2 + (n_chunks2 - 1) * chunk2
    pltpu.sync_copy(dst_ref.at[pl.ds(off_t2, chunk2)], dst2_v)
    pltpu.sync_copy(rows_v.at[pl.ds(0, chunk2)], agg_s.at[dst2_v], add=True)
    plsc.subcore_barrier()

    for k in range(n_stripes):
      pltpu.sync_copy(agg_s.at[pl.ds(rbase + k * stripe, stripe)],
                      deg_out_ref.at[c, pl.ds(rbase + k * stripe, stripe)])

  mesh = plsc.VectorSubcoreMesh(core_axis_name="c", subcore_axis_name="s")
  call = pl.kernel(
      body,
      out_type=(
          jax.ShapeDtypeStruct((2, n_pad, 128), jnp.float32),
          jax.ShapeDtypeStruct((2, n_pad, 128), jnp.float32),
      ),
      mesh=mesh,
      scratch_types=[
          pltpu.VMEM_SHARED((n_pad, 128), jnp.float32),
          pltpu.VMEM((2, 1, chunk), jnp.int32),
          pltpu.VMEM((2, 1, chunk), jnp.int32),
          pltpu.VMEM((chunk2,), jnp.int32),
          pltpu.VMEM((chunk, 128), jnp.float32),
          pltpu.VMEM((chunk2,), jnp.int32),
          pltpu.VMEM((chunk, 128), jnp.float32),
          pltpu.SemaphoreType.DMA,
          pltpu.SemaphoreType.DMA,
      ],
      name="sc_edge_agg",
  )
  src2_3d = src2.reshape(-1, 1, chunk)
  dst_3d = dst.reshape(-1, 1, chunk)
  return call(x2, src2_3d, dst_3d, dst, zeros_c, ones_c)


def _tc_body(aggm_ref, deg_ref, batch_ref, wg_ref, wl_ref, bl_ref,
             hn_ref, h_ref, out_ref, acc, cnt, *, n_g):
  i = pl.program_id(0)
  dg = deg_ref[...]
  a0 = aggm_ref[0] / dg
  a1 = aggm_ref[1] / dg
  a = jnp.concatenate([a0, a1], axis=1)
  hn = jnp.maximum(
      jnp.dot(a, wg_ref[...], preferred_element_type=jnp.float32), 0.0)
  hn_ref[...] = hn

  b = batch_ref[...]
  g = lax.broadcasted_iota(jnp.int32, b.shape, 1)
  oh = (b == g).astype(jnp.float32)
  part = lax.dot_general(oh, hn, (((0,), (0,)), ((), ())),
                         preferred_element_type=jnp.float32)
  pcnt = lax.dot_general(oh, jnp.ones_like(hn), (((0,), (0,)), ((), ())),
                         preferred_element_type=jnp.float32)

  @pl.when(i == 0)
  def _():
    acc[...] = jnp.zeros_like(acc)
    cnt[...] = jnp.zeros_like(cnt)

  acc[...] += part
  cnt[...] += pcnt

  @pl.when(i == pl.num_programs(0) - 1)
  def _():
    hh = acc[0:n_g, :] / jnp.maximum(cnt[0:n_g, :], 1.0)
    h_ref[...] = hh
    out_ref[...] = (
        jnp.dot(hh, wl_ref[...], preferred_element_type=jnp.float32)
        + bl_ref[0, :][None, :])


def kernel(x, edge_index, batch, Wg, Wl, bl):
  n, d = x.shape
  e = edge_index.shape[1]
  t = Wl.shape[1]
  n_g = 64
  assert d == 256

  rows_per_tile = 640
  n_pad = 16 * rows_per_tile
  chunk = 80
  n_chunks = e // (16 * chunk)

  x2 = x.reshape(2 * n, d // 2)
  src = edge_index[0]
  dst = edge_index[1]
  src2 = jnp.concatenate([src * 2, src * 2 + 1])
  zeros_c = jnp.zeros((64, 128), jnp.float32)
  ones_c = jnp.ones((40, 128), jnp.float32)

  aggm, degp = _sc_agg_call(x2, src2, dst, zeros_c, ones_c,
                            n_pad, rows_per_tile, chunk, n_chunks, e)

  deg = degp[0, :, 0] + degp[1, :, 0]
  deg_b = jnp.broadcast_to(jnp.maximum(deg, 1.0)[:, None], (n_pad, 128))

  batch_pad = jnp.pad(batch, (0, n_pad - n), constant_values=1 << 20)
  batch_b = jnp.broadcast_to(batch_pad[:, None], (n_pad, 128))
  bl8 = jnp.broadcast_to(bl[None, :], (8, t))

  blk = 1024
  grid = n_pad // blk

  hn_pad, h, out = pl.pallas_call(
      functools.partial(_tc_body, n_g=n_g),
      grid=(grid,),
      in_specs=[
          pl.BlockSpec((2, blk, 128), lambda i: (0, i, 0)),
          pl.BlockSpec((blk, 128), lambda i: (i, 0)),
          pl.BlockSpec((blk, 128), lambda i: (i, 0)),
          pl.BlockSpec((d, d), lambda i: (0, 0)),
          pl.BlockSpec((d, t), lambda i: (0, 0)),
          pl.BlockSpec((8, t), lambda i: (0, 0)),
      ],
      out_specs=[
          pl.BlockSpec((blk, d), lambda i: (i, 0)),
          pl.BlockSpec((n_g, d), lambda i: (0, 0)),
          pl.BlockSpec((n_g, t), lambda i: (0, 0)),
      ],
      out_shape=[
          jax.ShapeDtypeStruct((n_pad, d), jnp.float32),
          jax.ShapeDtypeStruct((n_g, d), jnp.float32),
          jax.ShapeDtypeStruct((n_g, t), jnp.float32),
      ],
      scratch_shapes=[
          pltpu.VMEM((128, d), jnp.float32),
          pltpu.VMEM((128, d), jnp.float32),
      ],
      name="tc_gnn_pool_head",
  )(aggm, deg_b, batch_b, Wg, Wl, bl8)

  h_n = hn_pad[:n]
  return (out, h_n, h)


# speedup vs baseline: 1.0164x; 1.0164x over previous
import functools

import jax
import jax.numpy as jnp
from jax import lax
from jax.experimental import pallas as pl
from jax.experimental.pallas import tpu as pltpu
from jax.experimental.pallas import tpu_sc as plsc


def _sc_agg_call(x2, src2, dst, zeros_c, ones_c,
                 n_pad, rows_per_tile, chunk, n_chunks, e):
  e_per_tile = chunk * n_chunks
  stripe = 64
  n_stripes = rows_per_tile // stripe
  chunk2 = 40
  n_chunks2 = (e // 2) // (16 * chunk2)

  def body(x2_ref, src2_ref, dst_ref, zeros_ref, ones_ref,
           out_ref, deg_out_ref, agg_s, idx_v, dst_v, dst2_v, rows_v,
           idx_w, dst_w, dst2_w, rows_w, sem, sem2, sem3, sem4):
    c = lax.axis_index("c")
    s = lax.axis_index("s")

    rbase = s * rows_per_tile
    for k in range(n_stripes):
      pltpu.sync_copy(zeros_ref, agg_s.at[pl.ds(rbase + k * stripe, stripe)])
    plsc.subcore_barrier()

    ebase = s * e_per_tile

    def pair_body(k, _):
      off_a = ebase + (2 * k) * chunk
      off_b = off_a + chunk
      pltpu.sync_copy(src2_ref.at[pl.ds(c * e + off_a, chunk)], idx_v)
      pltpu.sync_copy(dst_ref.at[pl.ds(off_a, chunk)], dst_v)
      ga = pltpu.async_copy(x2_ref.at[idx_v], rows_v, sem)
      pltpu.sync_copy(src2_ref.at[pl.ds(c * e + off_b, chunk)], idx_w)
      pltpu.sync_copy(dst_ref.at[pl.ds(off_b, chunk)], dst_w)
      ga.wait()
      gb = pltpu.async_copy(x2_ref.at[idx_w], rows_w, sem2)
      sa = pltpu.async_copy(rows_v, agg_s.at[dst_v], sem3, add=True)
      gb.wait()
      sb = pltpu.async_copy(rows_w, agg_s.at[dst_w], sem4, add=True)
      sa.wait()
      sb.wait()
      return 0
    lax.fori_loop(0, n_chunks // 2, pair_body, 0)
    off_t = ebase + (n_chunks - 1) * chunk
    pltpu.sync_copy(src2_ref.at[pl.ds(c * e + off_t, chunk)], idx_v)
    pltpu.sync_copy(dst_ref.at[pl.ds(off_t, chunk)], dst_v)
    pltpu.async_copy(x2_ref.at[idx_v], rows_v, sem).wait()
    pltpu.sync_copy(rows_v, agg_s.at[dst_v], add=True)
    plsc.subcore_barrier()

    for k in range(n_stripes):
      pltpu.sync_copy(agg_s.at[pl.ds(rbase + k * stripe, stripe)],
                      out_ref.at[c, pl.ds(rbase + k * stripe, stripe)])
    plsc.subcore_barrier()

    for k in range(n_stripes):
      pltpu.sync_copy(zeros_ref, agg_s.at[pl.ds(rbase + k * stripe, stripe)])
    pltpu.sync_copy(ones_ref, rows_v.at[pl.ds(0, chunk2)])
    plsc.subcore_barrier()

    ebase2 = (c * 16 + s) * (chunk2 * n_chunks2)

    def pair2_body(k, _):
      off_a = ebase2 + (2 * k) * chunk2
      off_b = off_a + chunk2
      pltpu.sync_copy(dst_ref.at[pl.ds(off_a, chunk2)], dst2_v)
      sa = pltpu.async_copy(rows_v.at[pl.ds(0, chunk2)],
                            agg_s.at[dst2_v], sem, add=True)
      pltpu.sync_copy(dst_ref.at[pl.ds(off_b, chunk2)], dst2_w)
      sa.wait()
      sb = pltpu.async_copy(rows_v.at[pl.ds(0, chunk2)],
                            agg_s.at[dst2_w], sem2, add=True)
      sb.wait()
      return 0
    lax.fori_loop(0, n_chunks2 // 2, pair2_body, 0)
    off_t2 = ebase2 + (n_chunks2 - 1) * chunk2
    pltpu.sync_copy(dst_ref.at[pl.ds(off_t2, chunk2)], dst2_v)
    pltpu.sync_copy(rows_v.at[pl.ds(0, chunk2)], agg_s.at[dst2_v], add=True)
    plsc.subcore_barrier()

    for k in range(n_stripes):
      pltpu.sync_copy(agg_s.at[pl.ds(rbase + k * stripe, stripe)],
                      deg_out_ref.at[c, pl.ds(rbase + k * stripe, stripe)])

  mesh = plsc.VectorSubcoreMesh(core_axis_name="c", subcore_axis_name="s")
  call = pl.kernel(
      body,
      out_type=(
          jax.ShapeDtypeStruct((2, n_pad, 128), jnp.float32),
          jax.ShapeDtypeStruct((2, n_pad, 128), jnp.float32),
      ),
      mesh=mesh,
      scratch_types=[
          pltpu.VMEM_SHARED((n_pad, 128), jnp.float32),
          pltpu.VMEM((chunk,), jnp.int32),
          pltpu.VMEM((chunk,), jnp.int32),
          pltpu.VMEM((chunk2,), jnp.int32),
          pltpu.VMEM((chunk, 128), jnp.float32),
          pltpu.VMEM((chunk,), jnp.int32),
          pltpu.VMEM((chunk,), jnp.int32),
          pltpu.VMEM((chunk2,), jnp.int32),
          pltpu.VMEM((chunk, 128), jnp.float32),
          pltpu.SemaphoreType.DMA,
          pltpu.SemaphoreType.DMA,
          pltpu.SemaphoreType.DMA,
          pltpu.SemaphoreType.DMA,
      ],
      name="sc_edge_agg",
  )
  return call(x2, src2, dst, zeros_c, ones_c)


def _tc_body(aggm_ref, deg_ref, batch_ref, wg_ref, wl_ref, bl_ref,
             hn_ref, h_ref, out_ref, acc, cnt, *, n_g):
  i = pl.program_id(0)
  dg = deg_ref[...]
  a0 = aggm_ref[0] / dg
  a1 = aggm_ref[1] / dg
  a = jnp.concatenate([a0, a1], axis=1)
  hn = jnp.maximum(
      jnp.dot(a, wg_ref[...], preferred_element_type=jnp.float32), 0.0)
  hn_ref[...] = hn

  b = batch_ref[...]
  g = lax.broadcasted_iota(jnp.int32, b.shape, 1)
  oh = (b == g).astype(jnp.float32)
  part = lax.dot_general(oh, hn, (((0,), (0,)), ((), ())),
                         preferred_element_type=jnp.float32)
  pcnt = lax.dot_general(oh, jnp.ones_like(hn), (((0,), (0,)), ((), ())),
                         preferred_element_type=jnp.float32)

  @pl.when(i == 0)
  def _():
    acc[...] = jnp.zeros_like(acc)
    cnt[...] = jnp.zeros_like(cnt)

  acc[...] += part
  cnt[...] += pcnt

  @pl.when(i == pl.num_programs(0) - 1)
  def _():
    hh = acc[0:n_g, :] / jnp.maximum(cnt[0:n_g, :], 1.0)
    h_ref[...] = hh
    out_ref[...] = (
        jnp.dot(hh, wl_ref[...], preferred_element_type=jnp.float32)
        + bl_ref[0, :][None, :])


def kernel(x, edge_index, batch, Wg, Wl, bl):
  n, d = x.shape
  e = edge_index.shape[1]
  t = Wl.shape[1]
  n_g = 64
  assert d == 256

  rows_per_tile = 640
  n_pad = 16 * rows_per_tile
  chunk = 80
  n_chunks = e // (16 * chunk)

  x2 = x.reshape(2 * n, d // 2)
  src = edge_index[0]
  dst = edge_index[1]
  src2 = jnp.concatenate([src * 2, src * 2 + 1])
  zeros_c = jnp.zeros((64, 128), jnp.float32)
  ones_c = jnp.ones((40, 128), jnp.float32)

  aggm, degp = _sc_agg_call(x2, src2, dst, zeros_c, ones_c,
                            n_pad, rows_per_tile, chunk, n_chunks, e)

  deg = degp[0, :, 0] + degp[1, :, 0]
  deg_b = jnp.broadcast_to(jnp.maximum(deg, 1.0)[:, None], (n_pad, 128))

  batch_pad = jnp.pad(batch, (0, n_pad - n), constant_values=1 << 20)
  batch_b = jnp.broadcast_to(batch_pad[:, None], (n_pad, 128))
  bl8 = jnp.broadcast_to(bl[None, :], (8, t))

  blk = 1024
  grid = n_pad // blk

  hn_pad, h, out = pl.pallas_call(
      functools.partial(_tc_body, n_g=n_g),
      grid=(grid,),
      in_specs=[
          pl.BlockSpec((2, blk, 128), lambda i: (0, i, 0)),
          pl.BlockSpec((blk, 128), lambda i: (i, 0)),
          pl.BlockSpec((blk, 128), lambda i: (i, 0)),
          pl.BlockSpec((d, d), lambda i: (0, 0)),
          pl.BlockSpec((d, t), lambda i: (0, 0)),
          pl.BlockSpec((8, t), lambda i: (0, 0)),
      ],
      out_specs=[
          pl.BlockSpec((blk, d), lambda i: (i, 0)),
          pl.BlockSpec((n_g, d), lambda i: (0, 0)),
          pl.BlockSpec((n_g, t), lambda i: (0, 0)),
      ],
      out_shape=[
          jax.ShapeDtypeStruct((n_pad, d), jnp.float32),
          jax.ShapeDtypeStruct((n_g, d), jnp.float32),
          jax.ShapeDtypeStruct((n_g, t), jnp.float32),
      ],
      scratch_shapes=[
          pltpu.VMEM((128, d), jnp.float32),
          pltpu.VMEM((128, d), jnp.float32),
      ],
      name="tc_gnn_pool_head",
  )(aggm, deg_b, batch_b, Wg, Wl, bl8)

  h_n = hn_pad[:n]
  return (out, h_n, h)
